# baseline (speedup 1.0000x reference)
import jax
import jax.numpy as jnp
from jax import lax
from jax.experimental import pallas as pl
from jax.experimental.pallas import tpu as pltpu

_ATM = 50000
_FROWS = 392


def _fused_body(idx_ref, pos_ref, out_ref, flg_ref, sem):
    cp = pltpu.make_async_copy(pos_ref, out_ref, sem)
    cp.start()
    flg_ref[...] = jnp.zeros_like(flg_ref)

    def body(k, carry):
        r = idx_ref[k]
        r = jnp.where(r < 0, r + _ATM, r)
        row = r // 128
        m = lax.broadcasted_iota(jnp.int32, (1, 128), 1) == (r % 128)
        flg_ref[pl.ds(row, 1), :] = flg_ref[pl.ds(row, 1), :] | m
        return carry

    lax.fori_loop(0, idx_ref.shape[0], body, 0)
    cp.wait()


def _fused(idx, pos_t):
    d, b, atm = pos_t.shape
    return pl.pallas_call(
        _fused_body,
        grid_spec=pltpu.PrefetchScalarGridSpec(
            num_scalar_prefetch=1,
            grid=(1,),
            in_specs=[pl.BlockSpec(memory_space=pl.ANY)],
            out_specs=[
                pl.BlockSpec(memory_space=pl.ANY),
                pl.BlockSpec((_FROWS, 128), lambda i, idx_ref: (0, 0)),
            ],
            scratch_shapes=[pltpu.SemaphoreType.DMA],
        ),
        out_shape=[
            jax.ShapeDtypeStruct((d, b, atm), pos_t.dtype),
            jax.ShapeDtypeStruct((_FROWS, 128), jnp.bool_),
        ],
    )(idx, pos_t)


def kernel(pos, idx):
    atm, dim = pos.shape[1], pos.shape[2]
    pos_ct, flags = _fused(idx, jnp.transpose(pos, (2, 0, 1)))
    pos_out = jnp.transpose(pos_ct, (1, 2, 0))
    msk = jnp.broadcast_to(flags.reshape(-1)[:atm][None, :, None], (1, atm, dim))
    return (pos_out, msk)


# speedup vs baseline: 15.5759x; 15.5759x over previous
import jax
import jax.numpy as jnp
from jax import lax
from jax.experimental import pallas as pl
from jax.experimental.pallas import tpu as pltpu

_ATM = 50000
_LBLK = 16768
_FROWS = 392


def _fused_body(idx_ref, pos_ref, out_ref, flg_ref):
    out_ref[...] = pos_ref[...]

    @pl.when(pl.program_id(0) == 0)
    def _():
        flg_ref[...] = jnp.zeros_like(flg_ref)

        def body(k, carry):
            r = idx_ref[k]
            r = jnp.where(r < 0, r + _ATM, r)
            row = r // 128
            m = lax.broadcasted_iota(jnp.int32, (1, 128), 1) == (r % 128)
            flg_ref[pl.ds(row, 1), :] = flg_ref[pl.ds(row, 1), :] | m
            return carry

        lax.fori_loop(0, idx_ref.shape[0], body, 0)


def _fused(idx, pos_t):
    d, b, atm = pos_t.shape
    return pl.pallas_call(
        _fused_body,
        grid_spec=pltpu.PrefetchScalarGridSpec(
            num_scalar_prefetch=1,
            grid=(pl.cdiv(atm, _LBLK),),
            in_specs=[pl.BlockSpec((d, b, _LBLK), lambda i, idx_ref: (0, 0, i))],
            out_specs=[
                pl.BlockSpec((d, b, _LBLK), lambda i, idx_ref: (0, 0, i)),
                pl.BlockSpec((_FROWS, 128), lambda i, idx_ref: (0, 0)),
            ],
        ),
        out_shape=[
            jax.ShapeDtypeStruct((d, b, atm), pos_t.dtype),
            jax.ShapeDtypeStruct((_FROWS, 128), jnp.bool_),
        ],
    )(idx, pos_t)


def kernel(pos, idx):
    atm, dim = pos.shape[1], pos.shape[2]
    pos_ct, flags = _fused(idx, jnp.transpose(pos, (2, 0, 1)))
    pos_out = jnp.transpose(pos_ct, (1, 2, 0))
    msk = jnp.broadcast_to(flags.reshape(-1)[:atm][None, :, None], (1, atm, dim))
    return (pos_out, msk)


# speedup vs baseline: 18.4472x; 1.1843x over previous
import jax
import jax.numpy as jnp
from jax import lax
from jax.experimental import pallas as pl
from jax.experimental.pallas import tpu as pltpu

_ATM = 50000
_LBLK = 25600
_FROWS = 392


def _fused_body(idx_ref, pos_ref, out_ref, flg_ref):
    out_ref[...] = pos_ref[...]

    @pl.when(pl.program_id(0) == pl.num_programs(0) - 1)
    def _():
        flg_ref[...] = jnp.zeros_like(flg_ref)

        def body(k, carry):
            r = idx_ref[k]
            r = jnp.where(r < 0, r + _ATM, r)
            row = r // 128
            m = lax.broadcasted_iota(jnp.int32, (1, 128), 1) == (r % 128)
            flg_ref[pl.ds(row, 1), :] = flg_ref[pl.ds(row, 1), :] | m
            return carry

        lax.fori_loop(0, idx_ref.shape[0], body, 0)


def _fused(idx, pos_t):
    d, b, atm = pos_t.shape
    return pl.pallas_call(
        _fused_body,
        grid_spec=pltpu.PrefetchScalarGridSpec(
            num_scalar_prefetch=1,
            grid=(pl.cdiv(atm, _LBLK),),
            in_specs=[pl.BlockSpec((d, b, _LBLK), lambda i, idx_ref: (0, 0, i))],
            out_specs=[
                pl.BlockSpec((d, b, _LBLK), lambda i, idx_ref: (0, 0, i)),
                pl.BlockSpec((_FROWS, 128), lambda i, idx_ref: (0, 0)),
            ],
        ),
        out_shape=[
            jax.ShapeDtypeStruct((d, b, atm), pos_t.dtype),
            jax.ShapeDtypeStruct((_FROWS, 128), jnp.bool_),
        ],
    )(idx, pos_t)


def kernel(pos, idx):
    atm, dim = pos.shape[1], pos.shape[2]
    pos_ct, flags = _fused(idx, jnp.transpose(pos, (2, 0, 1)))
    pos_out = jnp.transpose(pos_ct, (1, 2, 0))
    msk = jnp.broadcast_to(flags.reshape(-1)[:atm][None, :, None], (1, atm, dim))
    return (pos_out, msk)
